# Initial kernel scaffold; baseline (speedup 1.0000x reference)
import jax
import jax.numpy as jnp
from jax.experimental import pallas as pl


def kernel(x, table, pe):
    raise NotImplementedError("write your pallas kernel here")


# speedup vs baseline: 2.1164x; 2.1164x over previous
import functools

import jax
import jax.numpy as jnp
from jax import lax
from jax.experimental import pallas as pl
from jax.experimental.pallas import tpu as pltpu
from jax.experimental.pallas import tpu_sc as plsc

_LANES = 16


@functools.lru_cache(maxsize=None)
def _build(B, L, D, V):
    info = plsc.get_sparse_core_info()
    NC, NS = info.num_cores, info.num_subcores
    NW = NC * NS
    assert B % NW == 0
    seq_per_w = B // NW
    half = L // 2
    n_chunks = D // _LANES

    mesh = plsc.VectorSubcoreMesh(core_axis_name="c", subcore_axis_name="s")

    @functools.partial(
        pl.kernel,
        out_type=jax.ShapeDtypeStruct((B * L, D), jnp.float32),
        mesh=mesh,
        scratch_types=[
            pltpu.VMEM((2, half), jnp.int32),
            pltpu.VMEM((L, D), jnp.float32),
            pltpu.VMEM((L, D), jnp.float32),
            pltpu.SemaphoreType.DMA,
        ],
    )
    def emb(idx_hbm, pe_hbm, table_hbm, out_hbm, idx_v, buf, pe_v, sem):
        wid = lax.axis_index("s") * NC + lax.axis_index("c")
        pltpu.sync_copy(pe_hbm, pe_v)

        def seq_body(it, carry):
            base = (wid * seq_per_w + it) * L
            pltpu.sync_copy(idx_hbm.at[pl.ds(2 * (wid * seq_per_w + it), 2)],
                            idx_v)
            cp0 = pltpu.async_copy(
                table_hbm.at[idx_v.at[0]], buf.at[pl.ds(0, half)], sem)
            cp1 = pltpu.async_copy(
                table_hbm.at[idx_v.at[1]], buf.at[pl.ds(half, half)], sem)
            cp0.wait()
            cp1.wait()

            def add_row(r, c2):
                for cc in range(n_chunks):
                    sl = pl.ds(cc * _LANES, _LANES)
                    buf[r, sl] = buf[r, sl] + pe_v[r, sl]
                return c2

            lax.fori_loop(0, L, add_row, 0, unroll=2)
            pltpu.sync_copy(buf, out_hbm.at[pl.ds(base, L)])
            return carry

        lax.fori_loop(0, seq_per_w, seq_body, 0)

    return emb


def kernel(x, table, pe):
    B, L = x.shape
    V, D = table.shape
    idx = x.reshape(-1, L // 2).astype(jnp.int32)
    pe2 = pe[0, :L, :].astype(jnp.float32)
    out = _build(B, L, D, V)(idx, pe2, table)
    return out.reshape(B, L, D)
